# baseline (speedup 1.0000x reference)
import functools

import jax
import jax.numpy as jnp
from jax import lax
from jax.experimental import pallas as pl
from jax.experimental.pallas import tpu as pltpu
from jax.experimental.pallas import tpu_sc as plsc

N = 10000
E = 320000
D = 128
DH = 64
NPAD = 10240
ROW_BLK = 1000

NW = 32
EPW = E // NW
DEG_CHUNK = 2000

B = 80
RPT = N // 16


def _deg_body(col_hbm, out_hbm, colbuf, hist):
    c = lax.axis_index("c")
    s = lax.axis_index("s")
    wid = s * 2 + c
    zero16 = jnp.zeros((16,), jnp.float32)
    one16 = jnp.ones((16,), jnp.float32)

    def zb(i, carry):
        hist[pl.ds(i * 16, 16)] = zero16
        return carry

    lax.fori_loop(0, NPAD // 16, zb, 0)
    base = pl.multiple_of(wid * EPW, 8)

    def stage_fn(st, carry):
        off = pl.multiple_of(base + st * DEG_CHUNK, 8)
        pltpu.sync_copy(col_hbm.at[pl.ds(off, DEG_CHUNK)], colbuf)

        def inner(k, carry2):
            cv = colbuf[pl.ds(k * 16, 16)]
            plsc.addupdate_scatter(hist, [cv], one16)
            return carry2

        lax.fori_loop(0, DEG_CHUNK // 16, inner, 0)
        return carry

    lax.fori_loop(0, EPW // DEG_CHUNK, stage_fn, 0)
    pltpu.sync_copy(hist, out_hbm.at[wid])


_deg = pl.kernel(
    _deg_body,
    mesh=plsc.VectorSubcoreMesh(core_axis_name="c", subcore_axis_name="s"),
    compiler_params=pltpu.CompilerParams(needs_layout_passes=False, use_tc_tiling_on_sc=False),
    out_type=jax.ShapeDtypeStruct((NW, NPAD), jnp.float32),
    scratch_types=[
        pltpu.VMEM((DEG_CHUNK,), jnp.int32),
        pltpu.VMEM((NPAD,), jnp.float32),
    ],
)


def _dinv_body(parts_ref, dinv_ref):
    deg = jnp.sum(parts_ref[...], axis=0) + 1.0
    dinv_ref[...] = lax.rsqrt(deg)[:, None]


def _dinv(parts):
    return pl.pallas_call(
        _dinv_body,
        grid=(NPAD // 1024,),
        in_specs=[pl.BlockSpec((NW, 1024), lambda i: (0, i))],
        out_specs=pl.BlockSpec((1024, 1), lambda i: (i, 0)),
        out_shape=jax.ShapeDtypeStruct((NPAD, 1), jnp.float32),
    )(parts)


def _scale_body(x_ref, w_ref, dinv_ref, lo_ref, hi_ref):
    y = lax.dot_general(x_ref[...], w_ref[...], (((1,), (1,)), ((), ())),
                        preferred_element_type=jnp.float32)
    ys = y * dinv_ref[...]
    lo_ref[...] = ys[:, :DH]
    hi_ref[...] = ys[:, DH:]


def _scale(x, W, dinv):
    return pl.pallas_call(
        _scale_body,
        grid=(N // ROW_BLK,),
        in_specs=[
            pl.BlockSpec((ROW_BLK, D), lambda i: (i, 0)),
            pl.BlockSpec((D, D), lambda i: (0, 0)),
            pl.BlockSpec((ROW_BLK, 1), lambda i: (i, 0)),
        ],
        out_specs=[
            pl.BlockSpec((ROW_BLK, DH), lambda i: (i, 0)),
            pl.BlockSpec((ROW_BLK, DH), lambda i: (i, 0)),
        ],
        out_shape=[
            jax.ShapeDtypeStruct((N, DH), jnp.float32),
            jax.ShapeDtypeStruct((N, DH), jnp.float32),
        ],
    )(x, W, dinv)


EPT = E // 16
STG = 2000
NSTG = EPT // STG
NB_S = STG // B


def _scat_body(row_hbm, col_hbm, yslo_hbm, yshi_hbm, plo_hbm, phi_hbm,
               shys, shacc, rows_a, cols_a, rows_b, cols_b, g0, g1, g2, g3,
               gs0, gs1, gs2, gs3, ss0, ss1, ss2, ss3, isem_r, isem_c):
    c = lax.axis_index("c")
    s = lax.axis_index("s")
    r0 = s * RPT

    @pl.when(c == 0)
    def _():
        pltpu.sync_copy(yslo_hbm.at[pl.ds(r0, RPT)], shys.at[pl.ds(r0, RPT)])
        pltpu.sync_copy(yslo_hbm.at[pl.ds(r0, RPT)], shacc.at[pl.ds(r0, RPT)])

    @pl.when(c == 1)
    def _():
        pltpu.sync_copy(yshi_hbm.at[pl.ds(r0, RPT)], shys.at[pl.ds(r0, RPT)])
        pltpu.sync_copy(yshi_hbm.at[pl.ds(r0, RPT)], shacc.at[pl.ds(r0, RPT)])

    plsc.subcore_barrier()

    e0 = s * EPT

    gbufs = (g0, g1, g2, g3)
    gsems = (gs0, gs1, gs2, gs3)
    ssems = (ss0, ss1, ss2, ss3)

    def drain_gather(g_buf, g_sem):
        pltpu.make_async_copy(yslo_hbm.at[pl.ds(0, B)], g_buf, g_sem).wait()

    def drain_scatter(g_buf, s_sem):
        pltpu.make_async_copy(g_buf, shacc.at[pl.ds(0, B)], s_sem).wait()

    def prefetch(st, rows_v, cols_v):
        off = pl.multiple_of(e0 + st * STG, 8)
        pltpu.async_copy(row_hbm.at[pl.ds(off, STG)], rows_v, isem_r)
        pltpu.async_copy(col_hbm.at[pl.ds(off, STG)], cols_v, isem_c)

    def wait_prefetch(rows_v, cols_v):
        pltpu.make_async_copy(row_hbm.at[pl.ds(0, STG)], rows_v, isem_r).wait()
        pltpu.make_async_copy(col_hbm.at[pl.ds(0, STG)], cols_v, isem_c).wait()

    def do_stage(rows_v, cols_v):
        def gath(t, i):
            pltpu.async_copy(
                shys.at[rows_v.at[pl.ds(t * B, B)]], gbufs[i], gsems[i])

        def scat(t, i):
            pltpu.async_copy(
                gbufs[i], shacc.at[cols_v.at[pl.ds(t * B, B)]], ssems[i],
                add=True)

        for i in range(4):
            gath(i, i)

        def grp(j, carry):
            for i in range(4):
                drain_gather(gbufs[i], gsems[i])
                scat(4 * j + i, i)
            for i in range(4):
                drain_scatter(gbufs[i], ssems[i])
                t = 4 * j + 4 + i

                @pl.when(t < NB_S)
                def _():
                    gath(t, i)

            return carry

        lax.fori_loop(0, NB_S // 4, grp, 0)
        drain_gather(gbufs[0], gsems[0])
        scat(NB_S - 1, 0)
        drain_scatter(gbufs[0], ssems[0])

    off0 = pl.multiple_of(e0, 8)
    pltpu.sync_copy(row_hbm.at[pl.ds(off0, STG)], rows_a)
    pltpu.sync_copy(col_hbm.at[pl.ds(off0, STG)], cols_a)

    def stage_pair(j, carry):
        nxt = jnp.minimum(2 * j + 1, NSTG - 1)
        prefetch(nxt, rows_b, cols_b)
        do_stage(rows_a, cols_a)
        wait_prefetch(rows_b, cols_b)
        nxt2 = jnp.minimum(2 * j + 2, NSTG - 1)
        prefetch(nxt2, rows_a, cols_a)
        do_stage(rows_b, cols_b)
        wait_prefetch(rows_a, cols_a)
        return carry

    lax.fori_loop(0, NSTG // 2, stage_pair, 0)
    plsc.subcore_barrier()

    @pl.when(c == 0)
    def _():
        pltpu.sync_copy(shacc.at[pl.ds(r0, RPT)], plo_hbm.at[pl.ds(r0, RPT)])

    @pl.when(c == 1)
    def _():
        pltpu.sync_copy(shacc.at[pl.ds(r0, RPT)], phi_hbm.at[pl.ds(r0, RPT)])


_scatter = pl.kernel(
    _scat_body,
    mesh=plsc.VectorSubcoreMesh(core_axis_name="c", subcore_axis_name="s"),
    compiler_params=pltpu.CompilerParams(needs_layout_passes=False, use_tc_tiling_on_sc=False),
    out_type=(
        jax.ShapeDtypeStruct((N, DH), jnp.float32),
        jax.ShapeDtypeStruct((N, DH), jnp.float32),
    ),
    scratch_types=[
        pltpu.VMEM_SHARED((N, DH), jnp.float32),
        pltpu.VMEM_SHARED((N, DH), jnp.float32),
        pltpu.VMEM((STG,), jnp.int32),
        pltpu.VMEM((STG,), jnp.int32),
        pltpu.VMEM((STG,), jnp.int32),
        pltpu.VMEM((STG,), jnp.int32),
        pltpu.VMEM((B, DH), jnp.float32),
        pltpu.VMEM((B, DH), jnp.float32),
        pltpu.VMEM((B, DH), jnp.float32),
        pltpu.VMEM((B, DH), jnp.float32),
        pltpu.SemaphoreType.DMA,
        pltpu.SemaphoreType.DMA,
        pltpu.SemaphoreType.DMA,
        pltpu.SemaphoreType.DMA,
        pltpu.SemaphoreType.DMA,
        pltpu.SemaphoreType.DMA,
        pltpu.SemaphoreType.DMA,
        pltpu.SemaphoreType.DMA,
        pltpu.SemaphoreType.DMA,
        pltpu.SemaphoreType.DMA,
    ],
)


def _final_body(plo_ref, phi_ref, dinv_ref, b_ref, pos_ref, o_ref):
    agg = jnp.concatenate([plo_ref[...], phi_ref[...]], axis=1) * dinv_ref[...]
    o_ref[...] = agg + b_ref[...] + pos_ref[...]


def _final(plo, phi, dinv, b2, pos):
    return pl.pallas_call(
        _final_body,
        grid=(N // ROW_BLK,),
        in_specs=[
            pl.BlockSpec((ROW_BLK, DH), lambda i: (i, 0)),
            pl.BlockSpec((ROW_BLK, DH), lambda i: (i, 0)),
            pl.BlockSpec((ROW_BLK, 1), lambda i: (i, 0)),
            pl.BlockSpec((1, D), lambda i: (0, 0)),
            pl.BlockSpec((ROW_BLK, D), lambda i: (i, 0)),
        ],
        out_specs=pl.BlockSpec((ROW_BLK, D), lambda i: (i, 0)),
        out_shape=jax.ShapeDtypeStruct((N, D), jnp.float32),
    )(plo, phi, dinv, b2, pos)


def kernel(x, edge_index, W, b, pos_embedding):
    row = edge_index[0]
    col = edge_index[1]
    parts = _deg(col)
    dinv = _dinv(parts)
    yslo, yshi = _scale(x, W, dinv)
    plo, phi = _scatter(row, col, yslo, yshi)
    return _final(plo, phi, dinv, b.reshape(1, D), pos_embedding)


# speedup vs baseline: 1.1460x; 1.1460x over previous
import functools

import jax
import jax.numpy as jnp
from jax import lax
from jax.experimental import pallas as pl
from jax.experimental.pallas import tpu as pltpu
from jax.experimental.pallas import tpu_sc as plsc

N = 10000
E = 320000
D = 128
DH = 64
NPAD = 10240
ROW_BLK = 1024
GRID = NPAD // ROW_BLK

NW = 32
EPW = E // NW
DEG_CHUNK = 2000

B = 80
RPT = N // 16


def _deg_body(col_hbm, out_hbm, colbuf, hist):
    c = lax.axis_index("c")
    s = lax.axis_index("s")
    wid = s * 2 + c
    zero16 = jnp.zeros((16,), jnp.float32)
    one16 = jnp.ones((16,), jnp.float32)

    def zb(i, carry):
        hist[pl.ds(i * 16, 16)] = zero16
        return carry

    lax.fori_loop(0, NPAD // 16, zb, 0)
    base = pl.multiple_of(wid * EPW, 8)

    def stage_fn(st, carry):
        off = pl.multiple_of(base + st * DEG_CHUNK, 8)
        pltpu.sync_copy(col_hbm.at[pl.ds(off, DEG_CHUNK)], colbuf)

        def inner(k, carry2):
            cv = colbuf[pl.ds(k * 16, 16)]
            plsc.addupdate_scatter(hist, [cv], one16)
            return carry2

        lax.fori_loop(0, DEG_CHUNK // 16, inner, 0)
        return carry

    lax.fori_loop(0, EPW // DEG_CHUNK, stage_fn, 0)
    pltpu.sync_copy(hist, out_hbm.at[wid])


_deg = pl.kernel(
    _deg_body,
    mesh=plsc.VectorSubcoreMesh(core_axis_name="c", subcore_axis_name="s"),
    compiler_params=pltpu.CompilerParams(needs_layout_passes=False, use_tc_tiling_on_sc=False),
    out_type=jax.ShapeDtypeStruct((NW, NPAD), jnp.float32),
    scratch_types=[
        pltpu.VMEM((DEG_CHUNK,), jnp.int32),
        pltpu.VMEM((NPAD,), jnp.float32),
    ],
)


def _scale_body(parts_ref, x_ref, w_ref, lo_ref, hi_ref, dinv_ref):
    i = pl.program_id(0)
    deg = jnp.sum(parts_ref[:, pl.ds(i * ROW_BLK, ROW_BLK)], axis=0) + 1.0
    dinv = lax.rsqrt(deg)[:, None]
    dinv_ref[...] = dinv
    y = lax.dot_general(x_ref[...], w_ref[...], (((1,), (1,)), ((), ())),
                        preferred_element_type=jnp.float32)
    ys = y * dinv
    lo_ref[...] = ys[:, :DH]
    hi_ref[...] = ys[:, DH:]


def _scale(parts, x, W):
    return pl.pallas_call(
        _scale_body,
        grid=(GRID,),
        in_specs=[
            pl.BlockSpec((NW, NPAD), lambda i: (0, 0)),
            pl.BlockSpec((ROW_BLK, D), lambda i: (i, 0)),
            pl.BlockSpec((D, D), lambda i: (0, 0)),
        ],
        out_specs=[
            pl.BlockSpec((ROW_BLK, DH), lambda i: (i, 0)),
            pl.BlockSpec((ROW_BLK, DH), lambda i: (i, 0)),
            pl.BlockSpec((ROW_BLK, 1), lambda i: (i, 0)),
        ],
        out_shape=[
            jax.ShapeDtypeStruct((N, DH), jnp.float32),
            jax.ShapeDtypeStruct((N, DH), jnp.float32),
            jax.ShapeDtypeStruct((N, 1), jnp.float32),
        ],
    )(parts, x, W)


EPT = E // 16
STG = 2000
NSTG = EPT // STG
NB_S = STG // B


def _scat_body(row_hbm, col_hbm, yslo_hbm, yshi_hbm, plo_hbm, phi_hbm,
               shys, shacc, rows_a, cols_a, rows_b, cols_b, g0, g1,
               gs0, gs1, isem_r, isem_c):
    c = lax.axis_index("c")
    s = lax.axis_index("s")
    r0 = s * RPT

    @pl.when(c == 0)
    def _():
        pltpu.sync_copy(yslo_hbm.at[pl.ds(r0, RPT)], shys.at[pl.ds(r0, RPT)])
        pltpu.sync_copy(yslo_hbm.at[pl.ds(r0, RPT)], shacc.at[pl.ds(r0, RPT)])

    @pl.when(c == 1)
    def _():
        pltpu.sync_copy(yshi_hbm.at[pl.ds(r0, RPT)], shys.at[pl.ds(r0, RPT)])
        pltpu.sync_copy(yshi_hbm.at[pl.ds(r0, RPT)], shacc.at[pl.ds(r0, RPT)])

    plsc.subcore_barrier()

    e0 = s * EPT

    ga, gb = g0, g1
    gsem_a, gsem_b = gs0, gs1

    def drain(g_buf, g_sem):
        pltpu.make_async_copy(yslo_hbm.at[pl.ds(0, B)], g_buf, g_sem).wait()

    def prefetch(st, rows_v, cols_v):
        off = pl.multiple_of(e0 + st * STG, 8)
        pltpu.async_copy(row_hbm.at[pl.ds(off, STG)], rows_v, isem_r)
        pltpu.async_copy(col_hbm.at[pl.ds(off, STG)], cols_v, isem_c)

    def wait_prefetch(rows_v, cols_v):
        pltpu.make_async_copy(row_hbm.at[pl.ds(0, STG)], rows_v, isem_r).wait()
        pltpu.make_async_copy(col_hbm.at[pl.ds(0, STG)], cols_v, isem_c).wait()

    def do_stage(rows_v, cols_v):
        pltpu.async_copy(shys.at[rows_v.at[pl.ds(0, B)]], ga, gsem_a)

        def bat(k, carry):
            drain(ga, gsem_a)
            pltpu.async_copy(shys.at[rows_v.at[pl.ds((2 * k + 1) * B, B)]], gb, gsem_b)
            pltpu.sync_copy(ga, shacc.at[cols_v.at[pl.ds((2 * k) * B, B)]], add=True)
            drain(gb, gsem_b)
            pltpu.async_copy(shys.at[rows_v.at[pl.ds((2 * k + 2) * B, B)]], ga, gsem_a)
            pltpu.sync_copy(gb, shacc.at[cols_v.at[pl.ds((2 * k + 1) * B, B)]], add=True)
            return carry

        lax.fori_loop(0, (NB_S - 1) // 2, bat, 0)
        drain(ga, gsem_a)
        pltpu.sync_copy(ga, shacc.at[cols_v.at[pl.ds((NB_S - 1) * B, B)]], add=True)

    off0 = pl.multiple_of(e0, 8)
    pltpu.sync_copy(row_hbm.at[pl.ds(off0, STG)], rows_a)
    pltpu.sync_copy(col_hbm.at[pl.ds(off0, STG)], cols_a)

    def stage_pair(j, carry):
        nxt = jnp.minimum(2 * j + 1, NSTG - 1)
        prefetch(nxt, rows_b, cols_b)
        do_stage(rows_a, cols_a)
        wait_prefetch(rows_b, cols_b)
        nxt2 = jnp.minimum(2 * j + 2, NSTG - 1)
        prefetch(nxt2, rows_a, cols_a)
        do_stage(rows_b, cols_b)
        wait_prefetch(rows_a, cols_a)
        return carry

    lax.fori_loop(0, NSTG // 2, stage_pair, 0)
    plsc.subcore_barrier()

    @pl.when(c == 0)
    def _():
        pltpu.sync_copy(shacc.at[pl.ds(r0, RPT)], plo_hbm.at[pl.ds(r0, RPT)])

    @pl.when(c == 1)
    def _():
        pltpu.sync_copy(shacc.at[pl.ds(r0, RPT)], phi_hbm.at[pl.ds(r0, RPT)])


_scatter = pl.kernel(
    _scat_body,
    mesh=plsc.VectorSubcoreMesh(core_axis_name="c", subcore_axis_name="s"),
    compiler_params=pltpu.CompilerParams(needs_layout_passes=False, use_tc_tiling_on_sc=False),
    out_type=(
        jax.ShapeDtypeStruct((N, DH), jnp.float32),
        jax.ShapeDtypeStruct((N, DH), jnp.float32),
    ),
    scratch_types=[
        pltpu.VMEM_SHARED((N, DH), jnp.float32),
        pltpu.VMEM_SHARED((N, DH), jnp.float32),
        pltpu.VMEM((STG,), jnp.int32),
        pltpu.VMEM((STG,), jnp.int32),
        pltpu.VMEM((STG,), jnp.int32),
        pltpu.VMEM((STG,), jnp.int32),
        pltpu.VMEM((B, DH), jnp.float32),
        pltpu.VMEM((B, DH), jnp.float32),
        pltpu.SemaphoreType.DMA,
        pltpu.SemaphoreType.DMA,
        pltpu.SemaphoreType.DMA,
        pltpu.SemaphoreType.DMA,
    ],
)


def _final_body(plo_ref, phi_ref, dinv_ref, b_ref, pos_ref, o_ref):
    agg = jnp.concatenate([plo_ref[...], phi_ref[...]], axis=1) * dinv_ref[...]
    o_ref[...] = agg + b_ref[...] + pos_ref[...]


def _final(plo, phi, dinv, b2, pos):
    return pl.pallas_call(
        _final_body,
        grid=(GRID,),
        in_specs=[
            pl.BlockSpec((ROW_BLK, DH), lambda i: (i, 0)),
            pl.BlockSpec((ROW_BLK, DH), lambda i: (i, 0)),
            pl.BlockSpec((ROW_BLK, 1), lambda i: (i, 0)),
            pl.BlockSpec((1, D), lambda i: (0, 0)),
            pl.BlockSpec((ROW_BLK, D), lambda i: (i, 0)),
        ],
        out_specs=pl.BlockSpec((ROW_BLK, D), lambda i: (i, 0)),
        out_shape=jax.ShapeDtypeStruct((N, D), jnp.float32),
    )(plo, phi, dinv, b2, pos)


def kernel(x, edge_index, W, b, pos_embedding):
    row = edge_index[0]
    col = edge_index[1]
    parts = _deg(col)
    yslo, yshi, dinv = _scale(parts, x, W)
    plo, phi = _scatter(row, col, yslo, yshi)
    return _final(plo, phi, dinv, b.reshape(1, D), pos_embedding)
